# Initial kernel scaffold; baseline (speedup 1.0000x reference)
import jax
import jax.numpy as jnp
from jax.experimental import pallas as pl


def kernel(x, pe, edge_attr, gf, params, edge_index, batch_index):
    raise NotImplementedError("write your pallas kernel here")


# speedup vs baseline: 1.1155x; 1.1155x over previous
import functools

import jax
import jax.numpy as jnp
import numpy as np
from jax import lax
from jax.experimental import pallas as pl

HD = 64
NB = 128
LMAX = 1024
NEG = -1e9


def _dotT(a, b):
    return lax.dot_general(a, b, (((1,), (1,)), ((), ())),
                           preferred_element_type=jnp.float32)


def _dot(a, b):
    return jnp.dot(a, b, preferred_element_type=jnp.float32)


def _softmax(s):
    m = jnp.max(s, axis=-1, keepdims=True)
    e = jnp.exp(s - m)
    return e / jnp.sum(e, axis=-1, keepdims=True)


def _set_agg_body(xd_ref, mask_ref, ws_ref, bs_ref, seed_ref, out_ref):
    x = xd_ref[0]
    mk = mask_ref[0]
    W = ws_ref[...]
    bA = bs_ref[...]

    def b(i):
        return bA[i:i + 1, :]

    q = _dot(x, W[0]) + b(0)
    k = _dot(x, W[1]) + b(1)
    v = _dot(x, W[2]) + b(2)
    s = _dotT(q, k) * 0.125
    s = jnp.where(mk > 0, s, NEG)
    o = _dot(_softmax(s), v)
    o = _dot(o, W[3]) + b(3)
    h = o + x
    h = h + jnp.maximum(_dot(h, W[4]) + b(4), 0.0)

    xl = jnp.maximum(_dot(h, W[5]) + b(5), 0.0)

    sd = seed_ref[...]
    q2 = _dot(sd, W[6]) + b(6)
    k2 = _dot(xl, W[7]) + b(7)
    v2 = _dot(xl, W[8]) + b(8)
    s2 = _dotT(q2, k2) * 0.125
    s2 = jnp.where(mk > 0, s2, NEG)
    o2 = _dot(_softmax(s2), v2)
    o2 = _dot(o2, W[9]) + b(9)
    g = o2 + sd
    g = g + jnp.maximum(_dot(g, W[10]) + b(10), 0.0)

    v3 = _dot(g, W[11]) + b(11)
    o3 = _dot(v3, W[12]) + b(12)
    d = o3 + g
    d = d + jnp.maximum(_dot(d, W[13]) + b(13), 0.0)
    out_ref[0] = d


def _set_agg_pallas(xd, maskf, ws, bs, seed):
    return pl.pallas_call(
        _set_agg_body,
        grid=(NB,),
        in_specs=[
            pl.BlockSpec((1, LMAX, HD), lambda i: (i, 0, 0)),
            pl.BlockSpec((1, 1, LMAX), lambda i: (i, 0, 0)),
            pl.BlockSpec((14, HD, HD), lambda i: (0, 0, 0)),
            pl.BlockSpec((14, HD), lambda i: (0, 0)),
            pl.BlockSpec((1, HD), lambda i: (0, 0)),
        ],
        out_specs=pl.BlockSpec((1, 1, HD), lambda i: (i, 0, 0)),
        out_shape=jax.ShapeDtypeStruct((NB, 1, HD), jnp.float32),
    )(xd, maskf, ws, bs, seed)


def _stack_set_agg_params(p):
    a, m, d = p['enc']['attn'], p['pma_mab']['attn'], p['dec']['attn']
    ws = jnp.stack([
        a['Wq'], a['Wk'], a['Wv'], a['Wo'],
        p['enc']['lin']['W'], p['pma_lin']['W'],
        m['Wq'], m['Wk'], m['Wv'], m['Wo'],
        p['pma_mab']['lin']['W'],
        d['Wv'], d['Wo'], p['dec']['lin']['W'],
    ])
    bs = jnp.stack([
        a['bq'], a['bk'], a['bv'], a['bo'],
        p['enc']['lin']['b'], p['pma_lin']['b'],
        m['bq'], m['bk'], m['bv'], m['bo'],
        p['pma_mab']['lin']['b'],
        d['bv'], d['bo'], p['dec']['lin']['b'],
    ])
    return ws, bs, p['seed']


def _mha4_body(xd_ref, mask_ref, ws_ref, bs_ref, out_ref):
    x = xd_ref[0]
    mk = mask_ref[0]
    W = ws_ref[...]
    bA = bs_ref[...]

    def b(i):
        return bA[i:i + 1, :]

    q = _dot(x, W[0]) + b(0)
    k = _dot(x, W[1]) + b(1)
    v = _dot(x, W[2]) + b(2)
    outs = []
    for hh in range(4):
        sl = slice(hh * 16, (hh + 1) * 16)
        s = _dotT(q[:, sl], k[:, sl]) * 0.25
        s = jnp.where(mk > 0, s, NEG)
        outs.append(_dot(_softmax(s), v[:, sl]))
    o = jnp.concatenate(outs, axis=1)
    out_ref[0] = _dot(o, W[3]) + b(3)


def _mha4_pallas(xd, maskf, ws, bs):
    return pl.pallas_call(
        _mha4_body,
        grid=(NB,),
        in_specs=[
            pl.BlockSpec((1, LMAX, HD), lambda i: (i, 0, 0)),
            pl.BlockSpec((1, 1, LMAX), lambda i: (i, 0, 0)),
            pl.BlockSpec((4, HD, HD), lambda i: (0, 0, 0)),
            pl.BlockSpec((4, HD), lambda i: (0, 0)),
        ],
        out_specs=pl.BlockSpec((1, LMAX, HD), lambda i: (i, 0, 0)),
        out_shape=jax.ShapeDtypeStruct((NB, LMAX, HD), jnp.float32),
    )(xd, maskf, ws, bs)


def _bn(x, p):
    return x / np.sqrt(1.0 + 1e-5) * p['w'] + p['b']


def kernel(x, pe, edge_attr, gf, params, edge_index, batch_index):
    B = NB
    N = x.shape[0]

    edges = jnp.searchsorted(batch_index, jnp.arange(B + 1), side='left')
    starts = edges[:B]
    counts = (edges[1:] - edges[:B]).astype(jnp.int32)
    posL = jnp.arange(LMAX)
    gidx = jnp.minimum(starts[:, None] + posL[None, :], N - 1)
    maskf = (posL[None, :] < counts[:, None]).astype(jnp.float32)
    pos = jnp.minimum(jnp.arange(N) - starts[batch_index], LMAX - 1)
    flat_back = batch_index * LMAX + pos

    def to_dense(y):
        return y[gidx] * maskf[..., None]

    def seg_sum(y):
        return jnp.sum(to_dense(y), axis=1)

    mask3 = maskf[:, None, :]

    def set_agg(xn, p):
        ws, bs, seed = _stack_set_agg_params(p)
        r = _set_agg_pallas(to_dense(xn), mask3, ws, bs, seed)
        return jnp.nan_to_num(r.reshape(B, HD))

    xpe = _bn(pe, params['pe_bn'])
    xpe = xpe @ params['pe_lin']['W'] + params['pe_lin']['b']
    xn = jnp.concatenate([x, xpe], axis=1)
    xn = xn @ params['node_emb']['W'] + params['node_emb']['b']

    src = edge_index[0]
    dst = edge_index[1]

    def gps(xn, lp):
        Wc = params['edge_emb']['W'] @ lp['gine_lin_edge']['W']
        bc = (params['edge_emb']['b'] @ lp['gine_lin_edge']['W']
              + lp['gine_lin_edge']['b'])
        m = jax.nn.relu(xn[src] + edge_attr @ Wc + bc)
        agg = jax.ops.segment_sum(m, dst, num_segments=N)

        t = (1.0 + lp['gine_eps']) * xn + agg
        h = jax.nn.relu(t @ lp['nn1']['W'] + lp['nn1']['b'])
        h = h @ lp['nn2']['W'] + lp['nn2']['b']
        h = _bn(h + xn, lp['norm1'])

        wsA = jnp.stack([lp['attn']['Wq'], lp['attn']['Wk'],
                         lp['attn']['Wv'], lp['attn']['Wo']])
        bsA = jnp.stack([lp['attn']['bq'], lp['attn']['bk'],
                         lp['attn']['bv'], lp['attn']['bo']])
        had = _mha4_pallas(to_dense(xn), mask3, wsA, bsA)
        ha = had.reshape(B * LMAX, HD)[flat_back]
        ha = _bn(ha + xn, lp['norm2'])

        out = h + ha
        out = out + jax.nn.relu(out @ lp['mlp1']['W'] + lp['mlp1']['b']) \
            @ lp['mlp2']['W'] + lp['mlp2']['b']
        return _bn(out, lp['norm3'])

    def graph_norm(y, p):
        cnt = jnp.maximum(counts, 1).astype(y.dtype)[:, None]
        mean = seg_sum(y) / cnt
        out = y - p['alpha'] * mean[batch_index]
        var = seg_sum(out * out) / cnt
        return out / jnp.sqrt(var + 1e-5)[batch_index] * p['w'] + p['b']

    gr = set_agg(xn, params['aggr0'])
    for lp in params['layers']:
        xn = gps(xn, lp)
        xn = graph_norm(xn, lp['gn'])
        gr = gr + set_agg(xn, lp['aggr'])

    h = jax.nn.relu(gr @ params['lin1']['W'] + params['lin1']['b'])
    h = jax.nn.relu(h @ params['lin2']['W'] + params['lin2']['b'])
    h = jax.nn.relu(h @ params['lin3']['W'] + params['lin3']['b'])
    return h @ params['lin4']['W'] + params['lin4']['b']


# speedup vs baseline: 1.4759x; 1.3231x over previous
import functools

import jax
import jax.numpy as jnp
import numpy as np
from jax import lax
from jax.experimental import pallas as pl
from jax.experimental.pallas import tpu as pltpu
from jax.experimental.pallas import tpu_sc as plsc

HD = 64
NB = 128
LMAX = 1024
NEG = -1e9

ECH = 64
NCH = 784
TPW = ECH * NCH
EPAD = 16 * TPW
OWN = 25024
SPR = OWN + 64
ZROW = SPR // 16


def _dotT(a, b):
    return lax.dot_general(a, b, (((1,), (1,)), ((), ())),
                           preferred_element_type=jnp.float32)


def _dot(a, b):
    return jnp.dot(a, b, preferred_element_type=jnp.float32)


def _softmax(s):
    m = jnp.max(s, axis=-1, keepdims=True)
    e = jnp.exp(s - m)
    return e / jnp.sum(e, axis=-1, keepdims=True)


def _set_agg_body(xd_ref, mask_ref, ws_ref, bs_ref, seed_ref, out_ref):
    x = xd_ref[0]
    mk = mask_ref[0]
    W = ws_ref[...]
    bA = bs_ref[...]

    def b(i):
        return bA[i:i + 1, :]

    q = _dot(x, W[0]) + b(0)
    k = _dot(x, W[1]) + b(1)
    v = _dot(x, W[2]) + b(2)
    s = _dotT(q, k) * 0.125
    s = jnp.where(mk > 0, s, NEG)
    o = _dot(_softmax(s), v)
    o = _dot(o, W[3]) + b(3)
    h = o + x
    h = h + jnp.maximum(_dot(h, W[4]) + b(4), 0.0)

    xl = jnp.maximum(_dot(h, W[5]) + b(5), 0.0)

    sd = seed_ref[...]
    q2 = _dot(sd, W[6]) + b(6)
    k2 = _dot(xl, W[7]) + b(7)
    v2 = _dot(xl, W[8]) + b(8)
    s2 = _dotT(q2, k2) * 0.125
    s2 = jnp.where(mk > 0, s2, NEG)
    o2 = _dot(_softmax(s2), v2)
    o2 = _dot(o2, W[9]) + b(9)
    g = o2 + sd
    g = g + jnp.maximum(_dot(g, W[10]) + b(10), 0.0)

    v3 = _dot(g, W[11]) + b(11)
    o3 = _dot(v3, W[12]) + b(12)
    d = o3 + g
    d = d + jnp.maximum(_dot(d, W[13]) + b(13), 0.0)
    out_ref[0] = d


def _set_agg_pallas(xd, maskf, ws, bs, seed):
    return pl.pallas_call(
        _set_agg_body,
        grid=(NB,),
        in_specs=[
            pl.BlockSpec((1, LMAX, HD), lambda i: (i, 0, 0)),
            pl.BlockSpec((1, 1, LMAX), lambda i: (i, 0, 0)),
            pl.BlockSpec((14, HD, HD), lambda i: (0, 0, 0)),
            pl.BlockSpec((14, HD), lambda i: (0, 0)),
            pl.BlockSpec((1, HD), lambda i: (0, 0)),
        ],
        out_specs=pl.BlockSpec((1, 1, HD), lambda i: (i, 0, 0)),
        out_shape=jax.ShapeDtypeStruct((NB, 1, HD), jnp.float32),
    )(xd, maskf, ws, bs, seed)


def _stack_set_agg_params(p):
    a, m, d = p['enc']['attn'], p['pma_mab']['attn'], p['dec']['attn']
    ws = jnp.stack([
        a['Wq'], a['Wk'], a['Wv'], a['Wo'],
        p['enc']['lin']['W'], p['pma_lin']['W'],
        m['Wq'], m['Wk'], m['Wv'], m['Wo'],
        p['pma_mab']['lin']['W'],
        d['Wv'], d['Wo'], p['dec']['lin']['W'],
    ])
    bs = jnp.stack([
        a['bq'], a['bk'], a['bv'], a['bo'],
        p['enc']['lin']['b'], p['pma_lin']['b'],
        m['bq'], m['bk'], m['bv'], m['bo'],
        p['pma_mab']['lin']['b'],
        d['bv'], d['bo'], p['dec']['lin']['b'],
    ])
    return ws, bs, p['seed']


def _mha4_body(xd_ref, mask_ref, ws_ref, bs_ref, out_ref):
    x = xd_ref[0]
    mk = mask_ref[0]
    W = ws_ref[...]
    bA = bs_ref[...]

    def b(i):
        return bA[i:i + 1, :]

    q = _dot(x, W[0]) + b(0)
    k = _dot(x, W[1]) + b(1)
    v = _dot(x, W[2]) + b(2)
    outs = []
    for hh in range(4):
        sl = slice(hh * 16, (hh + 1) * 16)
        s = _dotT(q[:, sl], k[:, sl]) * 0.25
        s = jnp.where(mk > 0, s, NEG)
        outs.append(_dot(_softmax(s), v[:, sl]))
    o = jnp.concatenate(outs, axis=1)
    out_ref[0] = _dot(o, W[3]) + b(3)


def _mha4_pallas(xd, maskf, ws, bs):
    return pl.pallas_call(
        _mha4_body,
        grid=(NB,),
        in_specs=[
            pl.BlockSpec((1, LMAX, HD), lambda i: (i, 0, 0)),
            pl.BlockSpec((1, 1, LMAX), lambda i: (i, 0, 0)),
            pl.BlockSpec((4, HD, HD), lambda i: (0, 0, 0)),
            pl.BlockSpec((4, HD), lambda i: (0, 0)),
        ],
        out_specs=pl.BlockSpec((1, LMAX, HD), lambda i: (i, 0, 0)),
        out_shape=jax.ShapeDtypeStruct((NB, LMAX, HD), jnp.float32),
    )(xd, maskf, ws, bs)


def _edgec_body(ea_ref, w_ref, b_ref, out_ref):
    ea = ea_ref[...]
    W = w_ref[...]
    acc = b_ref[...]
    acc = acc + ea[:, 0:1] * W[0:1, :]
    acc = acc + ea[:, 1:2] * W[1:2, :]
    acc = acc + ea[:, 2:3] * W[2:3, :]
    acc = acc + ea[:, 3:4] * W[3:4, :]
    out_ref[...] = acc


_EBLK = 2048


def _edgec_pallas(ea_pad, Wc, bc):
    return pl.pallas_call(
        _edgec_body,
        grid=(EPAD // _EBLK,),
        in_specs=[
            pl.BlockSpec((_EBLK, 4), lambda i: (i, 0)),
            pl.BlockSpec((4, HD), lambda i: (0, 0)),
            pl.BlockSpec((1, HD), lambda i: (0, 0)),
        ],
        out_specs=pl.BlockSpec((_EBLK, HD), lambda i: (i, 0)),
        out_shape=jax.ShapeDtypeStruct((EPAD, HD), jnp.float32),
    )(ea_pad, Wc, bc.reshape(1, HD))


def _gine_sc(xn, src_pad, dst_pad, cpe):
    mesh = plsc.VectorSubcoreMesh(core_axis_name="c", subcore_axis_name="s")

    @functools.partial(
        pl.kernel,
        mesh=mesh,
        compiler_params=pltpu.CompilerParams(use_tc_tiling_on_sc=False),
        out_type=jax.ShapeDtypeStruct((2 * SPR, HD), jnp.float32),
        scratch_types=[
            pltpu.VMEM_SHARED((SPR, HD), jnp.float32),
            pltpu.VMEM((2, ECH), jnp.int32),
            pltpu.VMEM((2, ECH), jnp.int32),
            pltpu.VMEM((2, ECH), jnp.int32),
            pltpu.VMEM((2, ECH, HD), jnp.float32),
            pltpu.VMEM((2, ECH, HD), jnp.float32),
            pltpu.VMEM((ECH, HD), jnp.float32),
            pltpu.SemaphoreType.DMA,
            pltpu.SemaphoreType.DMA,
            pltpu.SemaphoreType.DMA,
            pltpu.SemaphoreType.DMA,
        ],
    )
    def k(x_hbm, src_hbm, dst_hbm, c_hbm, out_hbm,
          acc, srcb, dstb, idxb, xg, cb, mb,
          semx0, semx1, semc0, semc1):
        cid = lax.axis_index("c")
        sid = lax.axis_index("s")
        base_node = cid * OWN
        tile_e0 = sid * TPW
        semx = (semx0, semx1)
        semc = (semc0, semc1)

        def zrow(i, _):
            for g in range(4):
                mb[i, pl.ds(g * 16, 16)] = jnp.zeros((16,), jnp.float32)
            return 0

        lax.fori_loop(0, ECH, zrow, 0)
        zbase = sid * ZROW
        nfull = ZROW // ECH
        for t in range(nfull):
            pltpu.sync_copy(mb, acc.at[pl.ds(zbase + t * ECH, ECH)])
        rem = ZROW - nfull * ECH
        pltpu.sync_copy(mb.at[pl.ds(0, rem)],
                        acc.at[pl.ds(zbase + nfull * ECH, rem)])
        plsc.subcore_barrier()

        def issue(jj, b):
            off = tile_e0 + jj * ECH
            pltpu.sync_copy(src_hbm.at[pl.ds(off, ECH)], srcb.at[b])
            pltpu.sync_copy(dst_hbm.at[pl.ds(off, ECH)], dstb.at[b])
            for g in range(ECH // 16):
                sl = pl.ds(g * 16, 16)
                d = dstb[b, sl]
                loc = d - base_node
                ok = (loc >= 0) & (loc < OWN)
                idxb[b, sl] = jnp.where(ok, loc, OWN)
            pltpu.async_copy(x_hbm.at[srcb.at[b]], xg.at[b], semx[b])
            pltpu.async_copy(c_hbm.at[pl.ds(off, ECH)], cb.at[b], semc[b])

        def waitproc(jj, b):
            off = tile_e0 + jj * ECH
            pltpu.make_async_copy(x_hbm.at[srcb.at[b]], xg.at[b],
                                  semx[b]).wait()
            pltpu.make_async_copy(c_hbm.at[pl.ds(off, ECH)], cb.at[b],
                                  semc[b]).wait()

            def mrow(i, _):
                for g in range(4):
                    sl = pl.ds(g * 16, 16)
                    mb[i, sl] = jnp.maximum(xg[b, i, sl] + cb[b, i, sl], 0.0)
                return 0

            lax.fori_loop(0, ECH, mrow, 0)
            pltpu.sync_copy(mb, acc.at[idxb.at[b]], add=True)

        issue(0, 0)

        def pair(p, _):
            for bb in range(2):
                jj = 2 * p + bb

                @pl.when(jj + 1 < NCH)
                def _():
                    issue(jj + 1, 1 - bb)

                waitproc(jj, bb)
            return 0

        lax.fori_loop(0, NCH // 2, pair, 0)

        plsc.subcore_barrier()
        pltpu.sync_copy(acc.at[pl.ds(sid * ZROW, ZROW)],
                        out_hbm.at[pl.ds(cid * SPR + sid * ZROW, ZROW)])

    return k(xn, src_pad, dst_pad, cpe)


def _bn(x, p):
    return x / np.sqrt(1.0 + 1e-5) * p['w'] + p['b']


def kernel(x, pe, edge_attr, gf, params, edge_index, batch_index):
    B = NB
    N = x.shape[0]

    edges = jnp.searchsorted(batch_index, jnp.arange(B + 1), side='left')
    starts = edges[:B]
    counts = (edges[1:] - edges[:B]).astype(jnp.int32)
    posL = jnp.arange(LMAX)
    gidx = jnp.minimum(starts[:, None] + posL[None, :], N - 1)
    maskf = (posL[None, :] < counts[:, None]).astype(jnp.float32)
    pos = jnp.minimum(jnp.arange(N) - starts[batch_index], LMAX - 1)
    flat_back = batch_index * LMAX + pos

    def to_dense(y):
        return y[gidx] * maskf[..., None]

    def seg_sum(y):
        return jnp.sum(to_dense(y), axis=1)

    mask3 = maskf[:, None, :]

    def set_agg(xn, p):
        ws, bs, seed = _stack_set_agg_params(p)
        r = _set_agg_pallas(to_dense(xn), mask3, ws, bs, seed)
        return jnp.nan_to_num(r.reshape(B, HD))

    xpe = _bn(pe, params['pe_bn'])
    xpe = xpe @ params['pe_lin']['W'] + params['pe_lin']['b']
    xn = jnp.concatenate([x, xpe], axis=1)
    xn = xn @ params['node_emb']['W'] + params['node_emb']['b']

    E = edge_attr.shape[0]
    src_pad = jnp.concatenate(
        [edge_index[0], jnp.zeros((EPAD - E,), jnp.int32)])
    dst_pad = jnp.concatenate(
        [edge_index[1], jnp.full((EPAD - E,), -1, jnp.int32)])
    ea_pad = jnp.concatenate(
        [edge_attr, jnp.zeros((EPAD - E, 4), jnp.float32)], axis=0)

    def gps(xn, lp):
        Wc = params['edge_emb']['W'] @ lp['gine_lin_edge']['W']
        bc = (params['edge_emb']['b'] @ lp['gine_lin_edge']['W']
              + lp['gine_lin_edge']['b'])
        cpe = _edgec_pallas(ea_pad, Wc, bc)
        aggp = _gine_sc(xn, src_pad, dst_pad, cpe)
        agg = jnp.concatenate([aggp[:OWN], aggp[SPR:SPR + OWN]], axis=0)[:N]

        t = (1.0 + lp['gine_eps']) * xn + agg
        h = jax.nn.relu(t @ lp['nn1']['W'] + lp['nn1']['b'])
        h = h @ lp['nn2']['W'] + lp['nn2']['b']
        h = _bn(h + xn, lp['norm1'])

        wsA = jnp.stack([lp['attn']['Wq'], lp['attn']['Wk'],
                         lp['attn']['Wv'], lp['attn']['Wo']])
        bsA = jnp.stack([lp['attn']['bq'], lp['attn']['bk'],
                         lp['attn']['bv'], lp['attn']['bo']])
        had = _mha4_pallas(to_dense(xn), mask3, wsA, bsA)
        ha = had.reshape(B * LMAX, HD)[flat_back]
        ha = _bn(ha + xn, lp['norm2'])

        out = h + ha
        out = out + jax.nn.relu(out @ lp['mlp1']['W'] + lp['mlp1']['b']) \
            @ lp['mlp2']['W'] + lp['mlp2']['b']
        return _bn(out, lp['norm3'])

    def graph_norm(y, p):
        cnt = jnp.maximum(counts, 1).astype(y.dtype)[:, None]
        mean = seg_sum(y) / cnt
        out = y - p['alpha'] * mean[batch_index]
        var = seg_sum(out * out) / cnt
        return out / jnp.sqrt(var + 1e-5)[batch_index] * p['w'] + p['b']

    gr = set_agg(xn, params['aggr0'])
    for lp in params['layers']:
        xn = gps(xn, lp)
        xn = graph_norm(xn, lp['gn'])
        gr = gr + set_agg(xn, lp['aggr'])

    h = jax.nn.relu(gr @ params['lin1']['W'] + params['lin1']['b'])
    h = jax.nn.relu(h @ params['lin2']['W'] + params['lin2']['b'])
    h = jax.nn.relu(h @ params['lin3']['W'] + params['lin3']['b'])
    return h @ params['lin4']['W'] + params['lin4']['b']


# speedup vs baseline: 2.5675x; 1.7396x over previous
import functools

import jax
import jax.numpy as jnp
import numpy as np
from jax import lax
from jax.experimental import pallas as pl
from jax.experimental.pallas import tpu as pltpu
from jax.experimental.pallas import tpu_sc as plsc

HD = 64
NB = 128
LMAX = 1024
NEG = -1e9

ECH = 64
NCH = 784
TPW = ECH * NCH
EPAD = 16 * TPW
OWN = 25024
SPR = OWN + 64
ZROW = SPR // 16


def _dotT(a, b):
    return lax.dot_general(a, b, (((1,), (1,)), ((), ())),
                           preferred_element_type=jnp.float32)


def _dot(a, b):
    return jnp.dot(a, b, preferred_element_type=jnp.float32)


def _softmax(s):
    m = jnp.max(s, axis=-1, keepdims=True)
    e = jnp.exp(s - m)
    return e / jnp.sum(e, axis=-1, keepdims=True)


def _set_agg_body(xd_ref, mask_ref, ws_ref, bs_ref, seed_ref, out_ref):
    x = xd_ref[0]
    mk = mask_ref[0]
    W = ws_ref[...]
    bA = bs_ref[...]

    def b(i):
        return bA[i:i + 1, :]

    q = _dot(x, W[0]) + b(0)
    k = _dot(x, W[1]) + b(1)
    v = _dot(x, W[2]) + b(2)
    s = _dotT(q, k) * 0.125
    s = jnp.where(mk > 0, s, NEG)
    o = _dot(_softmax(s), v)
    o = _dot(o, W[3]) + b(3)
    h = o + x
    h = h + jnp.maximum(_dot(h, W[4]) + b(4), 0.0)

    xl = jnp.maximum(_dot(h, W[5]) + b(5), 0.0)

    sd = seed_ref[...]
    q2 = _dot(sd, W[6]) + b(6)
    k2 = _dot(xl, W[7]) + b(7)
    v2 = _dot(xl, W[8]) + b(8)
    s2 = _dotT(q2, k2) * 0.125
    s2 = jnp.where(mk > 0, s2, NEG)
    o2 = _dot(_softmax(s2), v2)
    o2 = _dot(o2, W[9]) + b(9)
    g = o2 + sd
    g = g + jnp.maximum(_dot(g, W[10]) + b(10), 0.0)

    v3 = _dot(g, W[11]) + b(11)
    o3 = _dot(v3, W[12]) + b(12)
    d = o3 + g
    d = d + jnp.maximum(_dot(d, W[13]) + b(13), 0.0)
    out_ref[0] = d


def _set_agg_pallas(L, xd, maskf, ws, bs, seed):
    return pl.pallas_call(
        _set_agg_body,
        grid=(NB,),
        in_specs=[
            pl.BlockSpec((1, L, HD), lambda i: (i, 0, 0)),
            pl.BlockSpec((1, 1, L), lambda i: (i, 0, 0)),
            pl.BlockSpec((14, HD, HD), lambda i: (0, 0, 0)),
            pl.BlockSpec((14, HD), lambda i: (0, 0)),
            pl.BlockSpec((1, HD), lambda i: (0, 0)),
        ],
        out_specs=pl.BlockSpec((1, 1, HD), lambda i: (i, 0, 0)),
        out_shape=jax.ShapeDtypeStruct((NB, 1, HD), jnp.float32),
    )(xd, maskf, ws, bs, seed)


def _stack_set_agg_params(p):
    a, m, d = p['enc']['attn'], p['pma_mab']['attn'], p['dec']['attn']
    ws = jnp.stack([
        a['Wq'], a['Wk'], a['Wv'], a['Wo'],
        p['enc']['lin']['W'], p['pma_lin']['W'],
        m['Wq'], m['Wk'], m['Wv'], m['Wo'],
        p['pma_mab']['lin']['W'],
        d['Wv'], d['Wo'], p['dec']['lin']['W'],
    ])
    bs = jnp.stack([
        a['bq'], a['bk'], a['bv'], a['bo'],
        p['enc']['lin']['b'], p['pma_lin']['b'],
        m['bq'], m['bk'], m['bv'], m['bo'],
        p['pma_mab']['lin']['b'],
        d['bv'], d['bo'], p['dec']['lin']['b'],
    ])
    return ws, bs, p['seed']


def _mha4_body(xd_ref, mask_ref, ws_ref, bs_ref, out_ref):
    x = xd_ref[0]
    mk = mask_ref[0]
    W = ws_ref[...]
    bA = bs_ref[...]

    def b(i):
        return bA[i:i + 1, :]

    q = _dot(x, W[0]) + b(0)
    k = _dot(x, W[1]) + b(1)
    v = _dot(x, W[2]) + b(2)
    outs = []
    for hh in range(4):
        sl = slice(hh * 16, (hh + 1) * 16)
        s = _dotT(q[:, sl], k[:, sl]) * 0.25
        s = jnp.where(mk > 0, s, NEG)
        outs.append(_dot(_softmax(s), v[:, sl]))
    o = jnp.concatenate(outs, axis=1)
    out_ref[0] = _dot(o, W[3]) + b(3)


def _mha4_pallas(L, xd, maskf, ws, bs):
    return pl.pallas_call(
        _mha4_body,
        grid=(NB,),
        in_specs=[
            pl.BlockSpec((1, L, HD), lambda i: (i, 0, 0)),
            pl.BlockSpec((1, 1, L), lambda i: (i, 0, 0)),
            pl.BlockSpec((4, HD, HD), lambda i: (0, 0, 0)),
            pl.BlockSpec((4, HD), lambda i: (0, 0)),
        ],
        out_specs=pl.BlockSpec((1, L, HD), lambda i: (i, 0, 0)),
        out_shape=jax.ShapeDtypeStruct((NB, L, HD), jnp.float32),
    )(xd, maskf, ws, bs)


def _edgec_body(ea_ref, w_ref, b_ref, out_ref):
    ea = ea_ref[...]
    W = w_ref[...]
    acc = b_ref[...]
    acc = acc + ea[:, 0:1] * W[0:1, :]
    acc = acc + ea[:, 1:2] * W[1:2, :]
    acc = acc + ea[:, 2:3] * W[2:3, :]
    acc = acc + ea[:, 3:4] * W[3:4, :]
    out_ref[...] = acc


_EBLK = 2048


def _edgec_pallas(ea_pad, Wc, bc):
    return pl.pallas_call(
        _edgec_body,
        grid=(EPAD // _EBLK,),
        in_specs=[
            pl.BlockSpec((_EBLK, 4), lambda i: (i, 0)),
            pl.BlockSpec((4, HD), lambda i: (0, 0)),
            pl.BlockSpec((1, HD), lambda i: (0, 0)),
        ],
        out_specs=pl.BlockSpec((_EBLK, HD), lambda i: (i, 0)),
        out_shape=jax.ShapeDtypeStruct((EPAD, HD), jnp.float32),
    )(ea_pad, Wc, bc.reshape(1, HD))


def _gine_sc(xn, src_pad, dst_pad, cpe):
    mesh = plsc.VectorSubcoreMesh(core_axis_name="c", subcore_axis_name="s")

    @functools.partial(
        pl.kernel,
        mesh=mesh,
        compiler_params=pltpu.CompilerParams(use_tc_tiling_on_sc=False),
        out_type=jax.ShapeDtypeStruct((2 * SPR, HD), jnp.float32),
        scratch_types=[
            pltpu.VMEM_SHARED((SPR, HD), jnp.float32),
            pltpu.VMEM((2, ECH), jnp.int32),
            pltpu.VMEM((2, ECH), jnp.int32),
            pltpu.VMEM((2, ECH), jnp.int32),
            pltpu.VMEM((2, ECH, HD), jnp.float32),
            pltpu.VMEM((2, ECH, HD), jnp.float32),
            pltpu.VMEM((ECH, HD), jnp.float32),
            pltpu.SemaphoreType.DMA,
            pltpu.SemaphoreType.DMA,
            pltpu.SemaphoreType.DMA,
            pltpu.SemaphoreType.DMA,
        ],
    )
    def k(x_hbm, src_hbm, dst_hbm, c_hbm, out_hbm,
          acc, srcb, dstb, idxb, xg, cb, mb,
          semx0, semx1, semc0, semc1):
        cid = lax.axis_index("c")
        sid = lax.axis_index("s")
        base_node = cid * OWN
        tile_e0 = sid * TPW
        semx = (semx0, semx1)
        semc = (semc0, semc1)

        def zrow(i, _):
            for g in range(4):
                mb[i, pl.ds(g * 16, 16)] = jnp.zeros((16,), jnp.float32)
            return 0

        lax.fori_loop(0, ECH, zrow, 0)
        zbase = sid * ZROW
        nfull = ZROW // ECH
        for t in range(nfull):
            pltpu.sync_copy(mb, acc.at[pl.ds(zbase + t * ECH, ECH)])
        rem = ZROW - nfull * ECH
        pltpu.sync_copy(mb.at[pl.ds(0, rem)],
                        acc.at[pl.ds(zbase + nfull * ECH, rem)])
        plsc.subcore_barrier()

        def issue(jj, b):
            off = tile_e0 + jj * ECH
            pltpu.sync_copy(src_hbm.at[pl.ds(off, ECH)], srcb.at[b])
            pltpu.sync_copy(dst_hbm.at[pl.ds(off, ECH)], dstb.at[b])
            for g in range(ECH // 16):
                sl = pl.ds(g * 16, 16)
                d = dstb[b, sl]
                loc = d - base_node
                ok = (loc >= 0) & (loc < OWN)
                idxb[b, sl] = jnp.where(ok, loc, OWN)
            pltpu.async_copy(x_hbm.at[srcb.at[b]], xg.at[b], semx[b])
            pltpu.async_copy(c_hbm.at[pl.ds(off, ECH)], cb.at[b], semc[b])

        def waitproc(jj, b):
            off = tile_e0 + jj * ECH
            pltpu.make_async_copy(x_hbm.at[srcb.at[b]], xg.at[b],
                                  semx[b]).wait()
            pltpu.make_async_copy(c_hbm.at[pl.ds(off, ECH)], cb.at[b],
                                  semc[b]).wait()

            def mrow(i, _):
                for g in range(4):
                    sl = pl.ds(g * 16, 16)
                    mb[i, sl] = jnp.maximum(xg[b, i, sl] + cb[b, i, sl], 0.0)
                return 0

            lax.fori_loop(0, ECH, mrow, 0)
            pltpu.sync_copy(mb, acc.at[idxb.at[b]], add=True)

        issue(0, 0)

        def pair(p, _):
            for bb in range(2):
                jj = 2 * p + bb

                @pl.when(jj + 1 < NCH)
                def _():
                    issue(jj + 1, 1 - bb)

                waitproc(jj, bb)
            return 0

        lax.fori_loop(0, NCH // 2, pair, 0)

        plsc.subcore_barrier()
        pltpu.sync_copy(acc.at[pl.ds(sid * ZROW, ZROW)],
                        out_hbm.at[pl.ds(cid * SPR + sid * ZROW, ZROW)])

    return k(xn, src_pad, dst_pad, cpe)


def _bn(x, p):
    return x / np.sqrt(1.0 + 1e-5) * p['w'] + p['b']


def _forward(L, x, pe, edge_attr, params, edge_index, batch_index):
    B = NB
    N = x.shape[0]

    bnds = jnp.searchsorted(batch_index, jnp.arange(B + 1), side='left')
    starts = bnds[:B]
    counts = (bnds[1:] - bnds[:B]).astype(jnp.int32)
    posL = jnp.arange(L)
    gidx = jnp.minimum(starts[:, None] + posL[None, :], N - 1)
    maskf = (posL[None, :] < counts[:, None]).astype(jnp.float32)
    pos = jnp.minimum(jnp.arange(N) - starts[batch_index], L - 1)
    flat_back = batch_index * L + pos

    def to_dense(y):
        return y[gidx] * maskf[..., None]

    def seg_sum(y):
        return jnp.sum(to_dense(y), axis=1)

    mask3 = maskf[:, None, :]

    def set_agg(xn, p):
        ws, bs, seed = _stack_set_agg_params(p)
        r = _set_agg_pallas(L, to_dense(xn), mask3, ws, bs, seed)
        return jnp.nan_to_num(r.reshape(B, HD))

    xpe = _bn(pe, params['pe_bn'])
    xpe = xpe @ params['pe_lin']['W'] + params['pe_lin']['b']
    xn = jnp.concatenate([x, xpe], axis=1)
    xn = xn @ params['node_emb']['W'] + params['node_emb']['b']

    E = edge_attr.shape[0]
    src_pad = jnp.concatenate(
        [edge_index[0], jnp.zeros((EPAD - E,), jnp.int32)])
    dst_pad = jnp.concatenate(
        [edge_index[1], jnp.full((EPAD - E,), -1, jnp.int32)])
    ea_pad = jnp.concatenate(
        [edge_attr, jnp.zeros((EPAD - E, 4), jnp.float32)], axis=0)

    def gps(xn, lp):
        Wc = params['edge_emb']['W'] @ lp['gine_lin_edge']['W']
        bc = (params['edge_emb']['b'] @ lp['gine_lin_edge']['W']
              + lp['gine_lin_edge']['b'])
        cpe = _edgec_pallas(ea_pad, Wc, bc)
        aggp = _gine_sc(xn, src_pad, dst_pad, cpe)
        agg = jnp.concatenate([aggp[:OWN], aggp[SPR:SPR + OWN]], axis=0)[:N]

        t = (1.0 + lp['gine_eps']) * xn + agg
        h = jax.nn.relu(t @ lp['nn1']['W'] + lp['nn1']['b'])
        h = h @ lp['nn2']['W'] + lp['nn2']['b']
        h = _bn(h + xn, lp['norm1'])

        wsA = jnp.stack([lp['attn']['Wq'], lp['attn']['Wk'],
                         lp['attn']['Wv'], lp['attn']['Wo']])
        bsA = jnp.stack([lp['attn']['bq'], lp['attn']['bk'],
                         lp['attn']['bv'], lp['attn']['bo']])
        had = _mha4_pallas(L, to_dense(xn), mask3, wsA, bsA)
        ha = had.reshape(B * L, HD)[flat_back]
        ha = _bn(ha + xn, lp['norm2'])

        out = h + ha
        out = out + jax.nn.relu(out @ lp['mlp1']['W'] + lp['mlp1']['b']) \
            @ lp['mlp2']['W'] + lp['mlp2']['b']
        return _bn(out, lp['norm3'])

    def graph_norm(y, p):
        cnt = jnp.maximum(counts, 1).astype(y.dtype)[:, None]
        mean = seg_sum(y) / cnt
        out = y - p['alpha'] * mean[batch_index]
        var = seg_sum(out * out) / cnt
        return out / jnp.sqrt(var + 1e-5)[batch_index] * p['w'] + p['b']

    gr = set_agg(xn, params['aggr0'])
    for lp in params['layers']:
        xn = gps(xn, lp)
        xn = graph_norm(xn, lp['gn'])
        gr = gr + set_agg(xn, lp['aggr'])

    h = jax.nn.relu(gr @ params['lin1']['W'] + params['lin1']['b'])
    h = jax.nn.relu(h @ params['lin2']['W'] + params['lin2']['b'])
    h = jax.nn.relu(h @ params['lin3']['W'] + params['lin3']['b'])
    return h @ params['lin4']['W'] + params['lin4']['b']


def kernel(x, pe, edge_attr, gf, params, edge_index, batch_index):
    bnds = jnp.searchsorted(batch_index, jnp.arange(NB + 1), side='left')
    cmax = jnp.max(bnds[1:] - bnds[:NB])
    args = (x, pe, edge_attr, params, edge_index, batch_index)
    return lax.cond(
        cmax > 512,
        lambda a: _forward(LMAX, *a),
        lambda a: _forward(512, *a),
        args,
    )


# speedup vs baseline: 2.8804x; 1.1219x over previous
import functools

import jax
import jax.numpy as jnp
import numpy as np
from jax import lax
from jax.experimental import pallas as pl
from jax.experimental.pallas import tpu as pltpu
from jax.experimental.pallas import tpu_sc as plsc

HD = 64
NB = 128
LMAX = 1024
NEG = -1e9

ECH = 64
NCH = 784
SB = 8
TPW = ECH * NCH
EPAD = 16 * TPW
OWN = 25024
SPR = OWN + 64
ZROW = SPR // 16


def _dotT(a, b):
    return lax.dot_general(a, b, (((1,), (1,)), ((), ())),
                           preferred_element_type=jnp.float32)


def _dot(a, b):
    return jnp.dot(a, b, preferred_element_type=jnp.float32)


def _softmax(s):
    m = jnp.max(s, axis=-1, keepdims=True)
    e = jnp.exp(s - m)
    return e / jnp.sum(e, axis=-1, keepdims=True)


def _set_agg_body(xd_ref, mask_ref, ws_ref, bs_ref, seed_ref, out_ref):
    x = xd_ref[0]
    mk = mask_ref[0]
    W = ws_ref[...]
    bA = bs_ref[...]

    def b(i):
        return bA[i:i + 1, :]

    q = _dot(x, W[0]) + b(0)
    k = _dot(x, W[1]) + b(1)
    v = _dot(x, W[2]) + b(2)
    s = _dotT(q, k) * 0.125
    s = jnp.where(mk > 0, s, NEG)
    o = _dot(_softmax(s), v)
    o = _dot(o, W[3]) + b(3)
    h = o + x
    h = h + jnp.maximum(_dot(h, W[4]) + b(4), 0.0)

    xl = jnp.maximum(_dot(h, W[5]) + b(5), 0.0)

    sd = seed_ref[...]
    q2 = _dot(sd, W[6]) + b(6)
    k2 = _dot(xl, W[7]) + b(7)
    v2 = _dot(xl, W[8]) + b(8)
    s2 = _dotT(q2, k2) * 0.125
    s2 = jnp.where(mk > 0, s2, NEG)
    o2 = _dot(_softmax(s2), v2)
    o2 = _dot(o2, W[9]) + b(9)
    g = o2 + sd
    g = g + jnp.maximum(_dot(g, W[10]) + b(10), 0.0)

    v3 = _dot(g, W[11]) + b(11)
    o3 = _dot(v3, W[12]) + b(12)
    d = o3 + g
    d = d + jnp.maximum(_dot(d, W[13]) + b(13), 0.0)
    out_ref[0] = d


def _set_agg_pallas(L, xd, maskf, ws, bs, seed):
    return pl.pallas_call(
        _set_agg_body,
        grid=(NB,),
        in_specs=[
            pl.BlockSpec((1, L, HD), lambda i: (i, 0, 0)),
            pl.BlockSpec((1, 1, L), lambda i: (i, 0, 0)),
            pl.BlockSpec((14, HD, HD), lambda i: (0, 0, 0)),
            pl.BlockSpec((14, HD), lambda i: (0, 0)),
            pl.BlockSpec((1, HD), lambda i: (0, 0)),
        ],
        out_specs=pl.BlockSpec((1, 1, HD), lambda i: (i, 0, 0)),
        out_shape=jax.ShapeDtypeStruct((NB, 1, HD), jnp.float32),
    )(xd, maskf, ws, bs, seed)


def _stack_set_agg_params(p):
    a, m, d = p['enc']['attn'], p['pma_mab']['attn'], p['dec']['attn']
    ws = jnp.stack([
        a['Wq'], a['Wk'], a['Wv'], a['Wo'],
        p['enc']['lin']['W'], p['pma_lin']['W'],
        m['Wq'], m['Wk'], m['Wv'], m['Wo'],
        p['pma_mab']['lin']['W'],
        d['Wv'], d['Wo'], p['dec']['lin']['W'],
    ])
    bs = jnp.stack([
        a['bq'], a['bk'], a['bv'], a['bo'],
        p['enc']['lin']['b'], p['pma_lin']['b'],
        m['bq'], m['bk'], m['bv'], m['bo'],
        p['pma_mab']['lin']['b'],
        d['bv'], d['bo'], p['dec']['lin']['b'],
    ])
    return ws, bs, p['seed']


def _mha4_body(xd_ref, mask_ref, ws_ref, bs_ref, out_ref):
    x = xd_ref[0]
    mk = mask_ref[0]
    W = ws_ref[...]
    bA = bs_ref[...]

    def b(i):
        return bA[i:i + 1, :]

    q = _dot(x, W[0]) + b(0)
    k = _dot(x, W[1]) + b(1)
    v = _dot(x, W[2]) + b(2)
    outs = []
    for hh in range(4):
        sl = slice(hh * 16, (hh + 1) * 16)
        s = _dotT(q[:, sl], k[:, sl]) * 0.25
        s = jnp.where(mk > 0, s, NEG)
        outs.append(_dot(_softmax(s), v[:, sl]))
    o = jnp.concatenate(outs, axis=1)
    out_ref[0] = _dot(o, W[3]) + b(3)


def _mha4_pallas(L, xd, maskf, ws, bs):
    return pl.pallas_call(
        _mha4_body,
        grid=(NB,),
        in_specs=[
            pl.BlockSpec((1, L, HD), lambda i: (i, 0, 0)),
            pl.BlockSpec((1, 1, L), lambda i: (i, 0, 0)),
            pl.BlockSpec((4, HD, HD), lambda i: (0, 0, 0)),
            pl.BlockSpec((4, HD), lambda i: (0, 0)),
        ],
        out_specs=pl.BlockSpec((1, L, HD), lambda i: (i, 0, 0)),
        out_shape=jax.ShapeDtypeStruct((NB, L, HD), jnp.float32),
    )(xd, maskf, ws, bs)


def _edgec_body(ea_ref, w_ref, b_ref, out_ref):
    ea = ea_ref[...]
    W = w_ref[...]
    acc = b_ref[...]
    acc = acc + ea[:, 0:1] * W[0:1, :]
    acc = acc + ea[:, 1:2] * W[1:2, :]
    acc = acc + ea[:, 2:3] * W[2:3, :]
    acc = acc + ea[:, 3:4] * W[3:4, :]
    out_ref[...] = acc


_EBLK = 2048


def _edgec_pallas(ea_pad, Wc, bc):
    return pl.pallas_call(
        _edgec_body,
        grid=(EPAD // _EBLK,),
        in_specs=[
            pl.BlockSpec((_EBLK, 4), lambda i: (i, 0)),
            pl.BlockSpec((4, HD), lambda i: (0, 0)),
            pl.BlockSpec((1, HD), lambda i: (0, 0)),
        ],
        out_specs=pl.BlockSpec((_EBLK, HD), lambda i: (i, 0)),
        out_shape=jax.ShapeDtypeStruct((EPAD, HD), jnp.float32),
    )(ea_pad, Wc, bc.reshape(1, HD))


def _gine_sc(xn, src_pad, dst_pad, cpe):
    mesh = plsc.VectorSubcoreMesh(core_axis_name="c", subcore_axis_name="s")

    @functools.partial(
        pl.kernel,
        mesh=mesh,
        compiler_params=pltpu.CompilerParams(use_tc_tiling_on_sc=False),
        out_type=jax.ShapeDtypeStruct((2 * SPR, HD), jnp.float32),
        scratch_types=[
            pltpu.VMEM_SHARED((SPR, HD), jnp.float32),
            pltpu.VMEM((2 * SB * ECH,), jnp.int32),
            pltpu.VMEM((2 * SB * ECH,), jnp.int32),
            pltpu.VMEM((2 * SB, ECH), jnp.int32),
            pltpu.VMEM((2, ECH, HD), jnp.float32),
            pltpu.VMEM((2, ECH, HD), jnp.float32),
            pltpu.VMEM((2, ECH, HD), jnp.float32),
            pltpu.SemaphoreType.DMA,
            pltpu.SemaphoreType.DMA,
            pltpu.SemaphoreType.DMA,
            pltpu.SemaphoreType.DMA,
            pltpu.SemaphoreType.DMA,
            pltpu.SemaphoreType.DMA,
        ],
    )
    def k(x_hbm, src_hbm, dst_hbm, c_hbm, out_hbm,
          acc, srcb, dstb, idxb, xg, cb, mb,
          semx0, semx1, semc0, semc1, sems0, sems1):
        cid = lax.axis_index("c")
        sid = lax.axis_index("s")
        base_node = cid * OWN
        tile_e0 = sid * TPW
        semx = (semx0, semx1)
        semc = (semc0, semc1)
        sems = (sems0, sems1)
        SBE = SB * ECH

        def zrow(i, _):
            for g in range(4):
                mb[0, i, pl.ds(g * 16, 16)] = jnp.zeros((16,), jnp.float32)
            return 0

        lax.fori_loop(0, ECH, zrow, 0)
        zbase = sid * ZROW
        nfull = ZROW // ECH
        for t in range(nfull):
            pltpu.sync_copy(mb.at[0], acc.at[pl.ds(zbase + t * ECH, ECH)])
        rem = ZROW - nfull * ECH
        pltpu.sync_copy(mb.at[0, pl.ds(0, rem)],
                        acc.at[pl.ds(zbase + nfull * ECH, rem)])
        plsc.subcore_barrier()

        def stage(blk):
            bp = lax.rem(blk, 2)
            off = tile_e0 + blk * SBE
            do = bp * SBE
            pltpu.sync_copy(src_hbm.at[pl.ds(off, SBE)],
                            srcb.at[pl.ds(do, SBE)])
            pltpu.sync_copy(dst_hbm.at[pl.ds(off, SBE)],
                            dstb.at[pl.ds(do, SBE)])
            for g in range(SBE // 16):
                d = dstb[pl.ds(do + g * 16, 16)]
                loc = d - base_node
                ok = (loc >= 0) & (loc < OWN)
                row = bp * SB + g // (ECH // 16)
                col = (g % (ECH // 16)) * 16
                idxb[row, pl.ds(col, 16)] = jnp.where(ok, loc, OWN)

        def gref(jj):
            bp = lax.rem(jj // SB, 2)
            ic = lax.rem(jj, SB)
            return srcb.at[pl.ds(bp * SBE + ic * ECH, ECH)]

        def issue(jj, b):
            @pl.when(lax.rem(jj, SB) == 0)
            def _():
                stage(jj // SB)

            off = tile_e0 + jj * ECH
            pltpu.async_copy(x_hbm.at[gref(jj)], xg.at[b], semx[b])
            pltpu.async_copy(c_hbm.at[pl.ds(off, ECH)], cb.at[b], semc[b])

        def waitproc(jj, b):
            off = tile_e0 + jj * ECH
            pltpu.make_async_copy(x_hbm.at[gref(jj)], xg.at[b],
                                  semx[b]).wait()
            pltpu.make_async_copy(c_hbm.at[pl.ds(off, ECH)], cb.at[b],
                                  semc[b]).wait()

            @pl.when(jj >= 2)
            def _():
                pltpu.make_async_copy(mb.at[b], acc.at[idxb.at[0]],
                                      sems[b]).wait()

            def mrow(i, _):
                for g in range(4):
                    sl = pl.ds(g * 16, 16)
                    mb[b, i, sl] = jnp.maximum(xg[b, i, sl] + cb[b, i, sl],
                                               0.0)
                return 0

            lax.fori_loop(0, ECH, mrow, 0)
            row = lax.rem(jj // SB, 2) * SB + lax.rem(jj, SB)
            pltpu.async_copy(mb.at[b], acc.at[idxb.at[row]], sems[b],
                             add=True)

        issue(0, 0)

        def pair(p, _):
            for bb in range(2):
                jj = 2 * p + bb

                @pl.when(jj + 1 < NCH)
                def _():
                    issue(jj + 1, 1 - bb)

                waitproc(jj, bb)
            return 0

        lax.fori_loop(0, NCH // 2, pair, 0)
        for b in range(2):
            pltpu.make_async_copy(mb.at[b], acc.at[idxb.at[0]],
                                  sems[b]).wait()

        plsc.subcore_barrier()
        pltpu.sync_copy(acc.at[pl.ds(sid * ZROW, ZROW)],
                        out_hbm.at[pl.ds(cid * SPR + sid * ZROW, ZROW)])

    return k(xn, src_pad, dst_pad, cpe)


def _bn(x, p):
    return x / np.sqrt(1.0 + 1e-5) * p['w'] + p['b']


def _forward(L, x, pe, edge_attr, params, edge_index, batch_index):
    B = NB
    N = x.shape[0]

    bnds = jnp.searchsorted(batch_index, jnp.arange(B + 1), side='left')
    starts = bnds[:B]
    counts = (bnds[1:] - bnds[:B]).astype(jnp.int32)
    posL = jnp.arange(L)
    gidx = jnp.minimum(starts[:, None] + posL[None, :], N - 1)
    maskf = (posL[None, :] < counts[:, None]).astype(jnp.float32)
    pos = jnp.minimum(jnp.arange(N) - starts[batch_index], L - 1)
    flat_back = batch_index * L + pos

    def to_dense(y):
        return y[gidx] * maskf[..., None]

    def seg_sum(y):
        return jnp.sum(to_dense(y), axis=1)

    mask3 = maskf[:, None, :]

    def set_agg(xn, p):
        ws, bs, seed = _stack_set_agg_params(p)
        r = _set_agg_pallas(L, to_dense(xn), mask3, ws, bs, seed)
        return jnp.nan_to_num(r.reshape(B, HD))

    xpe = _bn(pe, params['pe_bn'])
    xpe = xpe @ params['pe_lin']['W'] + params['pe_lin']['b']
    xn = jnp.concatenate([x, xpe], axis=1)
    xn = xn @ params['node_emb']['W'] + params['node_emb']['b']

    E = edge_attr.shape[0]
    src_pad = jnp.concatenate(
        [edge_index[0], jnp.zeros((EPAD - E,), jnp.int32)])
    dst_pad = jnp.concatenate(
        [edge_index[1], jnp.full((EPAD - E,), -1, jnp.int32)])
    ea_pad = jnp.concatenate(
        [edge_attr, jnp.zeros((EPAD - E, 4), jnp.float32)], axis=0)

    def gps(xn, lp):
        Wc = params['edge_emb']['W'] @ lp['gine_lin_edge']['W']
        bc = (params['edge_emb']['b'] @ lp['gine_lin_edge']['W']
              + lp['gine_lin_edge']['b'])
        cpe = _edgec_pallas(ea_pad, Wc, bc)
        aggp = _gine_sc(xn, src_pad, dst_pad, cpe)
        agg = jnp.concatenate([aggp[:OWN], aggp[SPR:SPR + OWN]], axis=0)[:N]

        t = (1.0 + lp['gine_eps']) * xn + agg
        h = jax.nn.relu(t @ lp['nn1']['W'] + lp['nn1']['b'])
        h = h @ lp['nn2']['W'] + lp['nn2']['b']
        h = _bn(h + xn, lp['norm1'])

        wsA = jnp.stack([lp['attn']['Wq'], lp['attn']['Wk'],
                         lp['attn']['Wv'], lp['attn']['Wo']])
        bsA = jnp.stack([lp['attn']['bq'], lp['attn']['bk'],
                         lp['attn']['bv'], lp['attn']['bo']])
        had = _mha4_pallas(L, to_dense(xn), mask3, wsA, bsA)
        ha = had.reshape(B * L, HD)[flat_back]
        ha = _bn(ha + xn, lp['norm2'])

        out = h + ha
        out = out + jax.nn.relu(out @ lp['mlp1']['W'] + lp['mlp1']['b']) \
            @ lp['mlp2']['W'] + lp['mlp2']['b']
        return _bn(out, lp['norm3'])

    def graph_norm(y, p):
        cnt = jnp.maximum(counts, 1).astype(y.dtype)[:, None]
        mean = seg_sum(y) / cnt
        out = y - p['alpha'] * mean[batch_index]
        var = seg_sum(out * out) / cnt
        return out / jnp.sqrt(var + 1e-5)[batch_index] * p['w'] + p['b']

    gr = set_agg(xn, params['aggr0'])
    for lp in params['layers']:
        xn = gps(xn, lp)
        xn = graph_norm(xn, lp['gn'])
        gr = gr + set_agg(xn, lp['aggr'])

    h = jax.nn.relu(gr @ params['lin1']['W'] + params['lin1']['b'])
    h = jax.nn.relu(h @ params['lin2']['W'] + params['lin2']['b'])
    h = jax.nn.relu(h @ params['lin3']['W'] + params['lin3']['b'])
    return h @ params['lin4']['W'] + params['lin4']['b']


def kernel(x, pe, edge_attr, gf, params, edge_index, batch_index):
    bnds = jnp.searchsorted(batch_index, jnp.arange(NB + 1), side='left')
    cmax = jnp.max(bnds[1:] - bnds[:NB])
    args = (x, pe, edge_attr, params, edge_index, batch_index)
    return lax.cond(
        cmax > 512,
        lambda a: _forward(LMAX, *a),
        lambda a: _forward(512, *a),
        args,
    )


# speedup vs baseline: 2.9522x; 1.0249x over previous
import functools

import jax
import jax.numpy as jnp
import numpy as np
from jax import lax
from jax.experimental import pallas as pl
from jax.experimental.pallas import tpu as pltpu
from jax.experimental.pallas import tpu_sc as plsc

HD = 64
NB = 128
LMAX = 1024
NEG = -1e9

ECH = 64
NCH = 784
SB = 8
TPW = ECH * NCH
EPAD = 16 * TPW
OWN = 25024
SPR = OWN + 64
ZROW = SPR // 16


def _dotT(a, b):
    return lax.dot_general(a, b, (((1,), (1,)), ((), ())),
                           preferred_element_type=jnp.float32)


def _dot(a, b):
    return jnp.dot(a, b, preferred_element_type=jnp.float32)


def _softmax(s):
    m = jnp.max(s, axis=-1, keepdims=True)
    e = jnp.exp(s - m)
    return e / jnp.sum(e, axis=-1, keepdims=True)


def _set_agg_body(xd_ref, mask_ref, ws_ref, bs_ref, seed_ref, out_ref):
    x = xd_ref[0]
    mk = mask_ref[0]
    W = ws_ref[...]
    bA = bs_ref[...]

    def b(i):
        return bA[i:i + 1, :]

    q = _dot(x, W[0]) + b(0)
    k = _dot(x, W[1]) + b(1)
    v = _dot(x, W[2]) + b(2)
    s = _dotT(q, k) * 0.125
    s = jnp.where(mk > 0, s, NEG)
    o = _dot(_softmax(s), v)
    o = _dot(o, W[3]) + b(3)
    h = o + x
    h = h + jnp.maximum(_dot(h, W[4]) + b(4), 0.0)

    xl = jnp.maximum(_dot(h, W[5]) + b(5), 0.0)

    sd = seed_ref[...]
    q2 = _dot(sd, W[6]) + b(6)
    k2 = _dot(xl, W[7]) + b(7)
    v2 = _dot(xl, W[8]) + b(8)
    s2 = _dotT(q2, k2) * 0.125
    s2 = jnp.where(mk > 0, s2, NEG)
    o2 = _dot(_softmax(s2), v2)
    o2 = _dot(o2, W[9]) + b(9)
    g = o2 + sd
    g = g + jnp.maximum(_dot(g, W[10]) + b(10), 0.0)

    v3 = _dot(g, W[11]) + b(11)
    o3 = _dot(v3, W[12]) + b(12)
    d = o3 + g
    d = d + jnp.maximum(_dot(d, W[13]) + b(13), 0.0)
    out_ref[0] = d


def _set_agg_pallas(L, xd, maskf, ws, bs, seed):
    return pl.pallas_call(
        _set_agg_body,
        grid=(NB,),
        in_specs=[
            pl.BlockSpec((1, L, HD), lambda i: (i, 0, 0)),
            pl.BlockSpec((1, 1, L), lambda i: (i, 0, 0)),
            pl.BlockSpec((14, HD, HD), lambda i: (0, 0, 0)),
            pl.BlockSpec((14, HD), lambda i: (0, 0)),
            pl.BlockSpec((1, HD), lambda i: (0, 0)),
        ],
        out_specs=pl.BlockSpec((1, 1, HD), lambda i: (i, 0, 0)),
        out_shape=jax.ShapeDtypeStruct((NB, 1, HD), jnp.float32),
    )(xd, maskf, ws, bs, seed)


def _stack_set_agg_params(p):
    a, m, d = p['enc']['attn'], p['pma_mab']['attn'], p['dec']['attn']
    ws = jnp.stack([
        a['Wq'], a['Wk'], a['Wv'], a['Wo'],
        p['enc']['lin']['W'], p['pma_lin']['W'],
        m['Wq'], m['Wk'], m['Wv'], m['Wo'],
        p['pma_mab']['lin']['W'],
        d['Wv'], d['Wo'], p['dec']['lin']['W'],
    ])
    bs = jnp.stack([
        a['bq'], a['bk'], a['bv'], a['bo'],
        p['enc']['lin']['b'], p['pma_lin']['b'],
        m['bq'], m['bk'], m['bv'], m['bo'],
        p['pma_mab']['lin']['b'],
        d['bv'], d['bo'], p['dec']['lin']['b'],
    ])
    return ws, bs, p['seed']


def _mha4_body(xd_ref, mask_ref, ws_ref, bs_ref, out_ref):
    x = xd_ref[0]
    mk = mask_ref[0]
    W = ws_ref[...]
    bA = bs_ref[...]

    def b(i):
        return bA[i:i + 1, :]

    q = _dot(x, W[0]) + b(0)
    k = _dot(x, W[1]) + b(1)
    v = _dot(x, W[2]) + b(2)
    outs = []
    for hh in range(4):
        sl = slice(hh * 16, (hh + 1) * 16)
        s = _dotT(q[:, sl], k[:, sl]) * 0.25
        s = jnp.where(mk > 0, s, NEG)
        outs.append(_dot(_softmax(s), v[:, sl]))
    o = jnp.concatenate(outs, axis=1)
    out_ref[0] = _dot(o, W[3]) + b(3)


def _mha4_pallas(L, xd, maskf, ws, bs):
    return pl.pallas_call(
        _mha4_body,
        grid=(NB,),
        in_specs=[
            pl.BlockSpec((1, L, HD), lambda i: (i, 0, 0)),
            pl.BlockSpec((1, 1, L), lambda i: (i, 0, 0)),
            pl.BlockSpec((4, HD, HD), lambda i: (0, 0, 0)),
            pl.BlockSpec((4, HD), lambda i: (0, 0)),
        ],
        out_specs=pl.BlockSpec((1, L, HD), lambda i: (i, 0, 0)),
        out_shape=jax.ShapeDtypeStruct((NB, L, HD), jnp.float32),
    )(xd, maskf, ws, bs)


def _edgec_body(ea_ref, w_ref, b_ref, out_ref):
    ea = ea_ref[...]
    W = w_ref[...]
    acc = b_ref[...]
    acc = acc + ea[:, 0:1] * W[0:1, :]
    acc = acc + ea[:, 1:2] * W[1:2, :]
    acc = acc + ea[:, 2:3] * W[2:3, :]
    acc = acc + ea[:, 3:4] * W[3:4, :]
    out_ref[...] = acc


_EBLK = 2048


def _edgec_pallas(ea_pad, Wc, bc):
    return pl.pallas_call(
        _edgec_body,
        grid=(EPAD // _EBLK,),
        in_specs=[
            pl.BlockSpec((_EBLK, 4), lambda i: (i, 0)),
            pl.BlockSpec((4, HD), lambda i: (0, 0)),
            pl.BlockSpec((1, HD), lambda i: (0, 0)),
        ],
        out_specs=pl.BlockSpec((_EBLK, HD), lambda i: (i, 0)),
        out_shape=jax.ShapeDtypeStruct((EPAD, HD), jnp.float32),
    )(ea_pad, Wc, bc.reshape(1, HD))


def _gine_sc(xn, src_pad, dst_pad, cpe):
    mesh = plsc.VectorSubcoreMesh(core_axis_name="c", subcore_axis_name="s")

    @functools.partial(
        pl.kernel,
        mesh=mesh,
        compiler_params=pltpu.CompilerParams(use_tc_tiling_on_sc=False),
        out_type=jax.ShapeDtypeStruct((2 * SPR, HD), jnp.float32),
        scratch_types=[
            pltpu.VMEM_SHARED((SPR, HD), jnp.float32),
            pltpu.VMEM((2 * SB * ECH,), jnp.int32),
            pltpu.VMEM((2 * SB * ECH,), jnp.int32),
            pltpu.VMEM((2 * SB, ECH), jnp.int32),
            pltpu.VMEM((2, ECH, HD), jnp.float32),
            pltpu.VMEM((2, ECH, HD), jnp.float32),
            pltpu.VMEM((2, ECH, HD), jnp.float32),
            pltpu.SemaphoreType.DMA,
            pltpu.SemaphoreType.DMA,
            pltpu.SemaphoreType.DMA,
            pltpu.SemaphoreType.DMA,
            pltpu.SemaphoreType.DMA,
            pltpu.SemaphoreType.DMA,
        ],
    )
    def k(x_hbm, src_hbm, dst_hbm, c_hbm, out_hbm,
          acc, srcb, dstb, idxb, xg, cb, mb,
          semx0, semx1, semc0, semc1, sems0, sems1):
        cid = lax.axis_index("c")
        sid = lax.axis_index("s")
        base_node = cid * OWN
        tile_e0 = sid * TPW
        semx = (semx0, semx1)
        semc = (semc0, semc1)
        sems = (sems0, sems1)
        SBE = SB * ECH

        def zrow(i, _):
            for g in range(4):
                mb[0, i, pl.ds(g * 16, 16)] = jnp.zeros((16,), jnp.float32)
            return 0

        lax.fori_loop(0, ECH, zrow, 0)
        zbase = sid * ZROW
        nfull = ZROW // ECH
        for t in range(nfull):
            pltpu.sync_copy(mb.at[0], acc.at[pl.ds(zbase + t * ECH, ECH)])
        rem = ZROW - nfull * ECH
        pltpu.sync_copy(mb.at[0, pl.ds(0, rem)],
                        acc.at[pl.ds(zbase + nfull * ECH, rem)])
        plsc.subcore_barrier()

        def stage(blk):
            bp = lax.rem(blk, 2)
            off = tile_e0 + blk * SBE
            do = bp * SBE
            pltpu.sync_copy(src_hbm.at[pl.ds(off, SBE)],
                            srcb.at[pl.ds(do, SBE)])
            pltpu.sync_copy(dst_hbm.at[pl.ds(off, SBE)],
                            dstb.at[pl.ds(do, SBE)])
            for g in range(SBE // 16):
                d = dstb[pl.ds(do + g * 16, 16)]
                loc = d - base_node
                ok = (loc >= 0) & (loc < OWN)
                row = bp * SB + g // (ECH // 16)
                col = (g % (ECH // 16)) * 16
                idxb[row, pl.ds(col, 16)] = jnp.where(ok, loc, OWN)

        def gref(jj):
            bp = lax.rem(jj // SB, 2)
            ic = lax.rem(jj, SB)
            return srcb.at[pl.ds(bp * SBE + ic * ECH, ECH)]

        def issue(jj, b):
            @pl.when(lax.rem(jj, SB) == 0)
            def _():
                stage(jj // SB)

            off = tile_e0 + jj * ECH
            pltpu.async_copy(x_hbm.at[gref(jj)], xg.at[b], semx[b])
            pltpu.async_copy(c_hbm.at[pl.ds(off, ECH)], cb.at[b], semc[b])

        def waitproc(jj, b):
            off = tile_e0 + jj * ECH
            pltpu.make_async_copy(x_hbm.at[gref(jj)], xg.at[b],
                                  semx[b]).wait()
            pltpu.make_async_copy(c_hbm.at[pl.ds(off, ECH)], cb.at[b],
                                  semc[b]).wait()

            @pl.when(jj >= 2)
            def _():
                pltpu.make_async_copy(mb.at[b], acc.at[idxb.at[0]],
                                      sems[b]).wait()

            def mrow(i, _):
                for g in range(4):
                    sl = pl.ds(g * 16, 16)
                    mb[b, i, sl] = jnp.maximum(xg[b, i, sl] + cb[b, i, sl],
                                               0.0)
                return 0

            lax.fori_loop(0, ECH, mrow, 0)
            row = lax.rem(jj // SB, 2) * SB + lax.rem(jj, SB)
            pltpu.async_copy(mb.at[b], acc.at[idxb.at[row]], sems[b],
                             add=True)

        issue(0, 0)

        def pair(p, _):
            for bb in range(2):
                jj = 2 * p + bb

                @pl.when(jj + 1 < NCH)
                def _():
                    issue(jj + 1, 1 - bb)

                waitproc(jj, bb)
            return 0

        lax.fori_loop(0, NCH // 2, pair, 0)
        for b in range(2):
            pltpu.make_async_copy(mb.at[b], acc.at[idxb.at[0]],
                                  sems[b]).wait()

        plsc.subcore_barrier()
        pltpu.sync_copy(acc.at[pl.ds(sid * ZROW, ZROW)],
                        out_hbm.at[pl.ds(cid * SPR + sid * ZROW, ZROW)])

    return k(xn, src_pad, dst_pad, cpe)


def _bn(x, p):
    return x / np.sqrt(1.0 + 1e-5) * p['w'] + p['b']


def _forward(L, x, pe, edge_attr, params, edge_index, batch_index):
    B = NB
    N = x.shape[0]

    bnds = jnp.searchsorted(batch_index, jnp.arange(B + 1), side='left')
    starts = bnds[:B]
    counts = (bnds[1:] - bnds[:B]).astype(jnp.int32)
    posL = jnp.arange(L)
    gidx = jnp.minimum(starts[:, None] + posL[None, :], N - 1)
    maskf = (posL[None, :] < counts[:, None]).astype(jnp.float32)
    pos = jnp.minimum(jnp.arange(N) - starts[batch_index], L - 1)
    flat_back = batch_index * L + pos

    def to_dense(y):
        return y[gidx] * maskf[..., None]

    def seg_sum(y):
        return jnp.sum(to_dense(y), axis=1)

    mask3 = maskf[:, None, :]

    def set_agg(xn, p):
        ws, bs, seed = _stack_set_agg_params(p)
        r = _set_agg_pallas(L, to_dense(xn), mask3, ws, bs, seed)
        return jnp.nan_to_num(r.reshape(B, HD))

    xpe = _bn(pe, params['pe_bn'])
    xpe = xpe @ params['pe_lin']['W'] + params['pe_lin']['b']
    xn = jnp.concatenate([x, xpe], axis=1)
    xn = xn @ params['node_emb']['W'] + params['node_emb']['b']

    E = edge_attr.shape[0]
    src_pad = jnp.concatenate(
        [edge_index[0], jnp.zeros((EPAD - E,), jnp.int32)])
    dst_pad = jnp.concatenate(
        [edge_index[1], jnp.full((EPAD - E,), -1, jnp.int32)])
    ea_pad = jnp.concatenate(
        [edge_attr, jnp.zeros((EPAD - E, 4), jnp.float32)], axis=0)

    def gps(xn, lp):
        Wc = params['edge_emb']['W'] @ lp['gine_lin_edge']['W']
        bc = (params['edge_emb']['b'] @ lp['gine_lin_edge']['W']
              + lp['gine_lin_edge']['b'])
        cpe = _edgec_pallas(ea_pad, Wc, bc)
        aggp = _gine_sc(xn, src_pad, dst_pad, cpe)
        agg = jnp.concatenate([aggp[:OWN], aggp[SPR:SPR + OWN]], axis=0)[:N]

        t = (1.0 + lp['gine_eps']) * xn + agg
        h = jax.nn.relu(t @ lp['nn1']['W'] + lp['nn1']['b'])
        h = h @ lp['nn2']['W'] + lp['nn2']['b']
        h = _bn(h + xn, lp['norm1'])

        wsA = jnp.stack([lp['attn']['Wq'], lp['attn']['Wk'],
                         lp['attn']['Wv'], lp['attn']['Wo']])
        bsA = jnp.stack([lp['attn']['bq'], lp['attn']['bk'],
                         lp['attn']['bv'], lp['attn']['bo']])
        had = _mha4_pallas(L, to_dense(xn), mask3, wsA, bsA)
        ha = had.reshape(B * L, HD)[flat_back]
        ha = _bn(ha + xn, lp['norm2'])

        out = h + ha
        out = out + jax.nn.relu(out @ lp['mlp1']['W'] + lp['mlp1']['b']) \
            @ lp['mlp2']['W'] + lp['mlp2']['b']
        return _bn(out, lp['norm3'])

    def graph_norm(y, p):
        cnt = jnp.maximum(counts, 1).astype(y.dtype)[:, None]
        dy = to_dense(y)
        mean = jnp.sum(dy, axis=1) / cnt
        out = y - p['alpha'] * mean[batch_index]
        dout = (dy - p['alpha'] * mean[:, None, :]) * maskf[..., None]
        var = jnp.sum(dout * dout, axis=1) / cnt
        return out / jnp.sqrt(var + 1e-5)[batch_index] * p['w'] + p['b']

    gr = set_agg(xn, params['aggr0'])
    for lp in params['layers']:
        xn = gps(xn, lp)
        xn = graph_norm(xn, lp['gn'])
        gr = gr + set_agg(xn, lp['aggr'])

    h = jax.nn.relu(gr @ params['lin1']['W'] + params['lin1']['b'])
    h = jax.nn.relu(h @ params['lin2']['W'] + params['lin2']['b'])
    h = jax.nn.relu(h @ params['lin3']['W'] + params['lin3']['b'])
    return h @ params['lin4']['W'] + params['lin4']['b']


def kernel(x, pe, edge_attr, gf, params, edge_index, batch_index):
    bnds = jnp.searchsorted(batch_index, jnp.arange(NB + 1), side='left')
    cmax = jnp.max(bnds[1:] - bnds[:NB])
    args = (x, pe, edge_attr, params, edge_index, batch_index)
    return lax.cond(
        cmax > 512,
        lambda a: _forward(LMAX, *a),
        lambda a: _forward(512, *a),
        args,
    )


# speedup vs baseline: 2.9822x; 1.0102x over previous
import functools

import jax
import jax.numpy as jnp
import numpy as np
from jax import lax
from jax.experimental import pallas as pl
from jax.experimental.pallas import tpu as pltpu
from jax.experimental.pallas import tpu_sc as plsc

HD = 64
NB = 128
LMAX = 1024
NEG = -1e9

ECH = 64
NCH = 784
SB = 8
TPW = ECH * NCH
EPAD = 16 * TPW
OWN = 25024
SPR = OWN + 64
ZROW = SPR // 16


def _dotT(a, b):
    return lax.dot_general(a, b, (((1,), (1,)), ((), ())),
                           preferred_element_type=jnp.float32)


def _dot(a, b):
    return jnp.dot(a, b, preferred_element_type=jnp.float32)


def _softmax(s):
    m = jnp.max(s, axis=-1, keepdims=True)
    e = jnp.exp(s - m)
    return e / jnp.sum(e, axis=-1, keepdims=True)


def _set_agg_body(xd_ref, mask_ref, ws_ref, bs_ref, seed_ref, out_ref):
    x = xd_ref[0]
    mk = mask_ref[0]
    W = ws_ref[...]
    bA = bs_ref[...]

    def b(i):
        return bA[i:i + 1, :]

    q = _dot(x, W[0]) + b(0)
    k = _dot(x, W[1]) + b(1)
    v = _dot(x, W[2]) + b(2)
    s = _dotT(q, k) * 0.125
    s = jnp.where(mk > 0, s, NEG)
    o = _dot(_softmax(s), v)
    o = _dot(o, W[3]) + b(3)
    h = o + x
    h = h + jnp.maximum(_dot(h, W[4]) + b(4), 0.0)

    xl = jnp.maximum(_dot(h, W[5]) + b(5), 0.0)

    sd = seed_ref[...]
    q2 = _dot(sd, W[6]) + b(6)
    k2 = _dot(xl, W[7]) + b(7)
    v2 = _dot(xl, W[8]) + b(8)
    s2 = _dotT(q2, k2) * 0.125
    s2 = jnp.where(mk > 0, s2, NEG)
    o2 = _dot(_softmax(s2), v2)
    o2 = _dot(o2, W[9]) + b(9)
    g = o2 + sd
    g = g + jnp.maximum(_dot(g, W[10]) + b(10), 0.0)

    v3 = _dot(g, W[11]) + b(11)
    o3 = _dot(v3, W[12]) + b(12)
    d = o3 + g
    d = d + jnp.maximum(_dot(d, W[13]) + b(13), 0.0)
    out_ref[0] = d


def _set_agg_pallas(L, xd, maskf, ws, bs, seed):
    return pl.pallas_call(
        _set_agg_body,
        grid=(NB,),
        in_specs=[
            pl.BlockSpec((1, L, HD), lambda i: (i, 0, 0)),
            pl.BlockSpec((1, 1, L), lambda i: (i, 0, 0)),
            pl.BlockSpec((14, HD, HD), lambda i: (0, 0, 0)),
            pl.BlockSpec((14, HD), lambda i: (0, 0)),
            pl.BlockSpec((1, HD), lambda i: (0, 0)),
        ],
        out_specs=pl.BlockSpec((1, 1, HD), lambda i: (i, 0, 0)),
        out_shape=jax.ShapeDtypeStruct((NB, 1, HD), jnp.float32),
    )(xd, maskf, ws, bs, seed)


def _stack_set_agg_params(p):
    a, m, d = p['enc']['attn'], p['pma_mab']['attn'], p['dec']['attn']
    ws = jnp.stack([
        a['Wq'], a['Wk'], a['Wv'], a['Wo'],
        p['enc']['lin']['W'], p['pma_lin']['W'],
        m['Wq'], m['Wk'], m['Wv'], m['Wo'],
        p['pma_mab']['lin']['W'],
        d['Wv'], d['Wo'], p['dec']['lin']['W'],
    ])
    bs = jnp.stack([
        a['bq'], a['bk'], a['bv'], a['bo'],
        p['enc']['lin']['b'], p['pma_lin']['b'],
        m['bq'], m['bk'], m['bv'], m['bo'],
        p['pma_mab']['lin']['b'],
        d['bv'], d['bo'], p['dec']['lin']['b'],
    ])
    return ws, bs, p['seed']


def _mha4_body(xd_ref, mask_ref, ws_ref, bs_ref, out_ref):
    x = xd_ref[0]
    mk = mask_ref[0]
    W = ws_ref[...]
    bA = bs_ref[...]

    def b(i):
        return bA[i:i + 1, :]

    q = _dot(x, W[0]) + b(0)
    k = _dot(x, W[1]) + b(1)
    v = _dot(x, W[2]) + b(2)
    outs = []
    for hh in range(4):
        sl = slice(hh * 16, (hh + 1) * 16)
        s = _dotT(q[:, sl], k[:, sl]) * 0.25
        s = jnp.where(mk > 0, s, NEG)
        outs.append(_dot(_softmax(s), v[:, sl]))
    o = jnp.concatenate(outs, axis=1)
    out_ref[0] = _dot(o, W[3]) + b(3)


def _mha4_pallas(L, xd, maskf, ws, bs):
    return pl.pallas_call(
        _mha4_body,
        grid=(NB,),
        in_specs=[
            pl.BlockSpec((1, L, HD), lambda i: (i, 0, 0)),
            pl.BlockSpec((1, 1, L), lambda i: (i, 0, 0)),
            pl.BlockSpec((4, HD, HD), lambda i: (0, 0, 0)),
            pl.BlockSpec((4, HD), lambda i: (0, 0)),
        ],
        out_specs=pl.BlockSpec((1, L, HD), lambda i: (i, 0, 0)),
        out_shape=jax.ShapeDtypeStruct((NB, L, HD), jnp.float32),
    )(xd, maskf, ws, bs)


def _edgec_body(ea_ref, w_ref, b_ref, out_ref):
    ea = ea_ref[...]
    W = w_ref[...]
    acc = b_ref[...]
    acc = acc + ea[:, 0:1] * W[0:1, :]
    acc = acc + ea[:, 1:2] * W[1:2, :]
    acc = acc + ea[:, 2:3] * W[2:3, :]
    acc = acc + ea[:, 3:4] * W[3:4, :]
    out_ref[...] = acc


_EBLK = 2048


def _edgec_pallas(ea_pad, Wc, bc):
    return pl.pallas_call(
        _edgec_body,
        grid=(EPAD // _EBLK,),
        in_specs=[
            pl.BlockSpec((_EBLK, 4), lambda i: (i, 0)),
            pl.BlockSpec((4, HD), lambda i: (0, 0)),
            pl.BlockSpec((1, HD), lambda i: (0, 0)),
        ],
        out_specs=pl.BlockSpec((_EBLK, HD), lambda i: (i, 0)),
        out_shape=jax.ShapeDtypeStruct((EPAD, HD), jnp.float32),
    )(ea_pad, Wc, bc.reshape(1, HD))


def _gine_sc(xn, src_pad, dst_pad, cpe):
    mesh = plsc.VectorSubcoreMesh(core_axis_name="c", subcore_axis_name="s")

    @functools.partial(
        pl.kernel,
        mesh=mesh,
        compiler_params=pltpu.CompilerParams(use_tc_tiling_on_sc=False),
        out_type=jax.ShapeDtypeStruct((2 * SPR, HD), jnp.float32),
        scratch_types=[
            pltpu.VMEM_SHARED((SPR, HD), jnp.float32),
            pltpu.VMEM((2 * SB * ECH,), jnp.int32),
            pltpu.VMEM((2 * SB * ECH,), jnp.int32),
            pltpu.VMEM((2 * SB, ECH), jnp.int32),
            pltpu.VMEM((2, ECH, HD), jnp.float32),
            pltpu.VMEM((2, ECH, HD), jnp.float32),
            pltpu.VMEM((2, ECH, HD), jnp.float32),
            pltpu.SemaphoreType.DMA,
            pltpu.SemaphoreType.DMA,
            pltpu.SemaphoreType.DMA,
            pltpu.SemaphoreType.DMA,
            pltpu.SemaphoreType.DMA,
            pltpu.SemaphoreType.DMA,
        ],
    )
    def k(x_hbm, src_hbm, dst_hbm, c_hbm, out_hbm,
          acc, srcb, dstb, idxb, xg, cb, mb,
          semx0, semx1, semc0, semc1, sems0, sems1):
        cid = lax.axis_index("c")
        sid = lax.axis_index("s")
        base_node = cid * OWN
        tile_e0 = sid * TPW
        semx = (semx0, semx1)
        semc = (semc0, semc1)
        sems = (sems0, sems1)
        SBE = SB * ECH

        def zrow(i, _):
            for g in range(4):
                mb[0, i, pl.ds(g * 16, 16)] = jnp.zeros((16,), jnp.float32)
            return 0

        lax.fori_loop(0, ECH, zrow, 0)
        zbase = sid * ZROW
        nfull = ZROW // ECH
        for t in range(nfull):
            pltpu.sync_copy(mb.at[0], acc.at[pl.ds(zbase + t * ECH, ECH)])
        rem = ZROW - nfull * ECH
        pltpu.sync_copy(mb.at[0, pl.ds(0, rem)],
                        acc.at[pl.ds(zbase + nfull * ECH, rem)])
        plsc.subcore_barrier()

        def stage(blk):
            bp = lax.rem(blk, 2)
            off = tile_e0 + blk * SBE
            do = bp * SBE
            pltpu.sync_copy(src_hbm.at[pl.ds(off, SBE)],
                            srcb.at[pl.ds(do, SBE)])
            pltpu.sync_copy(dst_hbm.at[pl.ds(off, SBE)],
                            dstb.at[pl.ds(do, SBE)])
            for g in range(SBE // 16):
                d = dstb[pl.ds(do + g * 16, 16)]
                loc = d - base_node
                ok = (loc >= 0) & (loc < OWN)
                row = bp * SB + g // (ECH // 16)
                col = (g % (ECH // 16)) * 16
                idxb[row, pl.ds(col, 16)] = jnp.where(ok, loc, OWN)

        def gref(jj):
            bp = lax.rem(jj // SB, 2)
            ic = lax.rem(jj, SB)
            return srcb.at[pl.ds(bp * SBE + ic * ECH, ECH)]

        def issue(jj, b):
            @pl.when(lax.rem(jj, SB) == 0)
            def _():
                stage(jj // SB)

            off = tile_e0 + jj * ECH
            pltpu.async_copy(x_hbm.at[gref(jj)], xg.at[b], semx[b])
            pltpu.async_copy(c_hbm.at[pl.ds(off, ECH)], cb.at[b], semc[b])

        def waitproc(jj, b):
            off = tile_e0 + jj * ECH
            pltpu.make_async_copy(x_hbm.at[gref(jj)], xg.at[b],
                                  semx[b]).wait()
            pltpu.make_async_copy(c_hbm.at[pl.ds(off, ECH)], cb.at[b],
                                  semc[b]).wait()

            @pl.when(jj >= 2)
            def _():
                pltpu.make_async_copy(mb.at[b], acc.at[idxb.at[0]],
                                      sems[b]).wait()

            def mrow(i, _):
                for g in range(4):
                    sl = pl.ds(g * 16, 16)
                    mb[b, i, sl] = jnp.maximum(xg[b, i, sl] + cb[b, i, sl],
                                               0.0)
                return 0

            lax.fori_loop(0, ECH, mrow, 0)
            row = lax.rem(jj // SB, 2) * SB + lax.rem(jj, SB)
            pltpu.async_copy(mb.at[b], acc.at[idxb.at[row]], sems[b],
                             add=True)

        issue(0, 0)

        def pair(p, _):
            for bb in range(2):
                jj = 2 * p + bb

                @pl.when(jj + 1 < NCH)
                def _():
                    issue(jj + 1, 1 - bb)

                waitproc(jj, bb)
            return 0

        lax.fori_loop(0, NCH // 2, pair, 0)
        for b in range(2):
            pltpu.make_async_copy(mb.at[b], acc.at[idxb.at[0]],
                                  sems[b]).wait()

        plsc.subcore_barrier()
        pltpu.sync_copy(acc.at[pl.ds(sid * ZROW, ZROW)],
                        out_hbm.at[pl.ds(cid * SPR + sid * ZROW, ZROW)])

    return k(xn, src_pad, dst_pad, cpe)


def _bn(x, p):
    return x / np.sqrt(1.0 + 1e-5) * p['w'] + p['b']


def _forward(L, x, pe, edge_attr, params, edge_index, batch_index):
    B = NB
    N = x.shape[0]

    bnds = jnp.searchsorted(batch_index, jnp.arange(B + 1), side='left')
    starts = bnds[:B]
    counts = (bnds[1:] - bnds[:B]).astype(jnp.int32)
    posL = jnp.arange(L)
    gidx = jnp.minimum(starts[:, None] + posL[None, :], N - 1)
    maskf = (posL[None, :] < counts[:, None]).astype(jnp.float32)
    pos = jnp.minimum(jnp.arange(N) - starts[batch_index], L - 1)
    flat_back = batch_index * L + pos

    def to_dense(y):
        return y[gidx] * maskf[..., None]

    def seg_sum(y):
        return jnp.sum(to_dense(y), axis=1)

    mask3 = maskf[:, None, :]

    def set_agg(xn, p):
        ws, bs, seed = _stack_set_agg_params(p)
        r = _set_agg_pallas(L, xn[gidx], mask3, ws, bs, seed)
        return jnp.nan_to_num(r.reshape(B, HD))

    xpe = _bn(pe, params['pe_bn'])
    xpe = xpe @ params['pe_lin']['W'] + params['pe_lin']['b']
    xn = jnp.concatenate([x, xpe], axis=1)
    xn = xn @ params['node_emb']['W'] + params['node_emb']['b']

    E = edge_attr.shape[0]
    src_pad = jnp.concatenate(
        [edge_index[0], jnp.zeros((EPAD - E,), jnp.int32)])
    dst_pad = jnp.concatenate(
        [edge_index[1], jnp.full((EPAD - E,), -1, jnp.int32)])
    ea_pad = jnp.concatenate(
        [edge_attr, jnp.zeros((EPAD - E, 4), jnp.float32)], axis=0)

    def gps(xn, lp):
        Wc = params['edge_emb']['W'] @ lp['gine_lin_edge']['W']
        bc = (params['edge_emb']['b'] @ lp['gine_lin_edge']['W']
              + lp['gine_lin_edge']['b'])
        cpe = _edgec_pallas(ea_pad, Wc, bc)
        aggp = _gine_sc(xn, src_pad, dst_pad, cpe)
        agg = jnp.concatenate([aggp[:OWN], aggp[SPR:SPR + OWN]], axis=0)[:N]

        t = (1.0 + lp['gine_eps']) * xn + agg
        h = jax.nn.relu(t @ lp['nn1']['W'] + lp['nn1']['b'])
        h = h @ lp['nn2']['W'] + lp['nn2']['b']
        h = _bn(h + xn, lp['norm1'])

        wsA = jnp.stack([lp['attn']['Wq'], lp['attn']['Wk'],
                         lp['attn']['Wv'], lp['attn']['Wo']])
        bsA = jnp.stack([lp['attn']['bq'], lp['attn']['bk'],
                         lp['attn']['bv'], lp['attn']['bo']])
        had = _mha4_pallas(L, xn[gidx], mask3, wsA, bsA)
        ha = had.reshape(B * L, HD)[flat_back]
        ha = _bn(ha + xn, lp['norm2'])

        out = h + ha
        out = out + jax.nn.relu(out @ lp['mlp1']['W'] + lp['mlp1']['b']) \
            @ lp['mlp2']['W'] + lp['mlp2']['b']
        return _bn(out, lp['norm3'])

    def graph_norm(y, p):
        cnt = jnp.maximum(counts, 1).astype(y.dtype)[:, None]
        dy = to_dense(y)
        mean = jnp.sum(dy, axis=1) / cnt
        out = y - p['alpha'] * mean[batch_index]
        dout = (dy - p['alpha'] * mean[:, None, :]) * maskf[..., None]
        var = jnp.sum(dout * dout, axis=1) / cnt
        return out / jnp.sqrt(var + 1e-5)[batch_index] * p['w'] + p['b']

    gr = set_agg(xn, params['aggr0'])
    for lp in params['layers']:
        xn = gps(xn, lp)
        xn = graph_norm(xn, lp['gn'])
        gr = gr + set_agg(xn, lp['aggr'])

    h = jax.nn.relu(gr @ params['lin1']['W'] + params['lin1']['b'])
    h = jax.nn.relu(h @ params['lin2']['W'] + params['lin2']['b'])
    h = jax.nn.relu(h @ params['lin3']['W'] + params['lin3']['b'])
    return h @ params['lin4']['W'] + params['lin4']['b']


def kernel(x, pe, edge_attr, gf, params, edge_index, batch_index):
    bnds = jnp.searchsorted(batch_index, jnp.arange(NB + 1), side='left')
    cmax = jnp.max(bnds[1:] - bnds[:NB])
    args = (x, pe, edge_attr, params, edge_index, batch_index)
    return lax.cond(
        cmax > 512,
        lambda a: _forward(LMAX, *a),
        lambda a: _forward(512, *a),
        args,
    )
